# baseline (speedup 1.0000x reference)
import jax
import jax.numpy as jnp
from jax import lax
from jax.experimental import pallas as pl
from jax.experimental.pallas import tpu as pltpu
from jax.experimental.pallas import tpu_sc as plsc

jax.config.update("jax_enable_x64", True)

NUM_CORES = 2
NUM_SUBCORES = 16
LANES = 16
NW = NUM_CORES * NUM_SUBCORES

N_NODES = 50000
N_ELEM = 800000

ROWS_PER_TILE = 3128
N_PAD = NUM_SUBCORES * ROWS_PER_TILE

CHUNK = 128
K_PER_BATCH = 8
BATCH = K_PER_BATCH * CHUNK
N_BATCH = 25
EPW = BATCH * N_BATCH
E_PAD = EPW * NW


def _sc_body(nA_hbm, nB_hbm, l_hbm, e_hbm, a_hbm, i_hbm, c_hbm, s_hbm,
             ux_hbm, uz_hbm, th_hbm, z_hbm,
             ox0, oz0, ot0, ox1, oz1, ot1,
             ux_sh, uz_sh, th_sh, fx_sh, fz_sh, ft_sh, stage,
             nA_v, nB_v, l_v, e_v, a_v, i_v, c_v, s_v,
             uxA0, uzA0, thA0, uxB0, uzB0, thB0,
             gxA0, gzA0, gtA0, gxB0, gzB0, gtB0,
             uxA1, uzA1, thA1, uxB1, uzB1, thB1,
             gxA1, gzA1, gtA1, gxB1, gzB1, gtB1,
             semL, semG0, semG1, semS0, semS1):
    i32 = jnp.int32
    cid = lax.axis_index("c")
    sid = lax.axis_index("s")
    wid = cid * i32(NUM_SUBCORES) + sid

    row0 = pl.multiple_of(sid * i32(ROWS_PER_TILE), 8)
    rows = pl.ds(row0, ROWS_PER_TILE)
    pltpu.sync_copy(ux_hbm.at[rows], stage)
    pltpu.sync_copy(stage, ux_sh.at[rows])
    pltpu.sync_copy(uz_hbm.at[rows], stage)
    pltpu.sync_copy(stage, uz_sh.at[rows])
    pltpu.sync_copy(th_hbm.at[rows], stage)
    pltpu.sync_copy(stage, th_sh.at[rows])
    pltpu.sync_copy(z_hbm.at[rows], stage)
    pltpu.sync_copy(stage, fx_sh.at[rows])
    pltpu.sync_copy(stage, fz_sh.at[rows])
    pltpu.sync_copy(stage, ft_sh.at[rows])
    plsc.subcore_barrier()

    ebase = wid * i32(EPW)
    rbase = wid * i32(EPW // CHUNK)

    def compute_chunk(j, uxA_v, uzA_v, thA_v, uxB_v, uzB_v, thB_v,
                      gxA_v, gzA_v, gtA_v, gxB_v, gzB_v, gtB_v):
        def step(i, carry3):
            sb = pl.ds(j * i32(CHUNK) + i * i32(LANES), LANES)
            sc = pl.ds(i * i32(LANES), LANES)
            uxA = uxA_v[sc]
            uzA = uzA_v[sc]
            thA = thA_v[sc]
            uxB = uxB_v[sc]
            uzB = uzB_v[sc]
            thB = thB_v[sc]
            el = l_v[sb]
            ee = e_v[sb]
            aa = a_v[sb]
            ii = i_v[sb]
            cc = c_v[sb]
            ss = s_v[sb]

            inv_l = 1.0 / el
            ea_l = ee * aa * inv_l
            ei_l = ee * ii * inv_l
            ei_l2 = ei_l * inv_l
            ei_l3 = ei_l2 * inv_l

            u_loc_d = cc * (uxA - uxB) + ss * (uzA - uzB)
            wA = cc * uzA - ss * uxA
            wB = cc * uzB - ss * uxB
            dw = wA - wB
            thAl = -thA
            thBl = -thB
            sth = thAl + thBl

            f0 = ea_l * u_loc_d
            f1 = 12.0 * ei_l3 * dw + 6.0 * ei_l2 * sth
            b_dw = 6.0 * ei_l2 * dw
            f2 = b_dw + 4.0 * ei_l * thAl + 2.0 * ei_l * thBl
            f5 = b_dw + 2.0 * ei_l * thAl + 4.0 * ei_l * thBl

            gAx = cc * f0 - ss * f1
            gAz = ss * f0 + cc * f1
            gxA_v[sc] = gAx
            gzA_v[sc] = gAz
            gtA_v[sc] = -f2
            gxB_v[sc] = -gAx
            gzB_v[sc] = -gAz
            gtB_v[sc] = -f5
            return carry3

        lax.fori_loop(i32(0), i32(CHUNK // LANES), step, i32(0),
                      unroll=False)

    def batch_body(bi, carry):
        eb = pl.multiple_of(ebase + bi * i32(BATCH), 8)
        rb = pl.multiple_of(rbase + bi * i32(K_PER_BATCH), 8)
        lds = [
            pltpu.async_copy(nA_hbm.at[pl.ds(rb, K_PER_BATCH)], nA_v, semL),
            pltpu.async_copy(nB_hbm.at[pl.ds(rb, K_PER_BATCH)], nB_v, semL),
            pltpu.async_copy(l_hbm.at[pl.ds(eb, BATCH)], l_v, semL),
            pltpu.async_copy(e_hbm.at[pl.ds(eb, BATCH)], e_v, semL),
            pltpu.async_copy(a_hbm.at[pl.ds(eb, BATCH)], a_v, semL),
            pltpu.async_copy(i_hbm.at[pl.ds(eb, BATCH)], i_v, semL),
            pltpu.async_copy(c_hbm.at[pl.ds(eb, BATCH)], c_v, semL),
            pltpu.async_copy(s_hbm.at[pl.ds(eb, BATCH)], s_v, semL),
        ]
        for cp in lds:
            cp.wait()

        def pair_body(p, carry2):
            j0 = p * i32(2)
            j1 = j0 + i32(1)
            idxA0 = nA_v.at[j0]
            idxB0 = nB_v.at[j0]
            idxA1 = nA_v.at[j1]
            idxB1 = nB_v.at[j1]
            ga = [
                pltpu.async_copy(ux_sh.at[idxA0], uxA0, semG0),
                pltpu.async_copy(uz_sh.at[idxA0], uzA0, semG0),
                pltpu.async_copy(th_sh.at[idxA0], thA0, semG0),
                pltpu.async_copy(ux_sh.at[idxB0], uxB0, semG0),
                pltpu.async_copy(uz_sh.at[idxB0], uzB0, semG0),
                pltpu.async_copy(th_sh.at[idxB0], thB0, semG0),
            ]
            gb = [
                pltpu.async_copy(ux_sh.at[idxA1], uxA1, semG1),
                pltpu.async_copy(uz_sh.at[idxA1], uzA1, semG1),
                pltpu.async_copy(th_sh.at[idxA1], thA1, semG1),
                pltpu.async_copy(ux_sh.at[idxB1], uxB1, semG1),
                pltpu.async_copy(uz_sh.at[idxB1], uzB1, semG1),
                pltpu.async_copy(th_sh.at[idxB1], thB1, semG1),
            ]
            for cp in ga:
                cp.wait()
            compute_chunk(j0, uxA0, uzA0, thA0, uxB0, uzB0, thB0,
                          gxA0, gzA0, gtA0, gxB0, gzB0, gtB0)
            sa = [
                pltpu.async_copy(gxA0, fx_sh.at[idxA0], semS0, add=True),
                pltpu.async_copy(gzA0, fz_sh.at[idxA0], semS0, add=True),
                pltpu.async_copy(gtA0, ft_sh.at[idxA0], semS0, add=True),
                pltpu.async_copy(gxB0, fx_sh.at[idxB0], semS0, add=True),
                pltpu.async_copy(gzB0, fz_sh.at[idxB0], semS0, add=True),
                pltpu.async_copy(gtB0, ft_sh.at[idxB0], semS0, add=True),
            ]
            for cp in gb:
                cp.wait()
            compute_chunk(j1, uxA1, uzA1, thA1, uxB1, uzB1, thB1,
                          gxA1, gzA1, gtA1, gxB1, gzB1, gtB1)
            sb_ = [
                pltpu.async_copy(gxA1, fx_sh.at[idxA1], semS1, add=True),
                pltpu.async_copy(gzA1, fz_sh.at[idxA1], semS1, add=True),
                pltpu.async_copy(gtA1, ft_sh.at[idxA1], semS1, add=True),
                pltpu.async_copy(gxB1, fx_sh.at[idxB1], semS1, add=True),
                pltpu.async_copy(gzB1, fz_sh.at[idxB1], semS1, add=True),
                pltpu.async_copy(gtB1, ft_sh.at[idxB1], semS1, add=True),
            ]
            for cp in sa:
                cp.wait()
            for cp in sb_:
                cp.wait()
            return carry2

        lax.fori_loop(i32(0), i32(K_PER_BATCH // 2), pair_body, i32(0),
                      unroll=False)
        return carry

    lax.fori_loop(i32(0), i32(N_BATCH), batch_body, i32(0), unroll=False)

    plsc.subcore_barrier()

    @pl.when(cid == i32(0))
    def _():
        pltpu.sync_copy(fx_sh.at[rows], stage)
        pltpu.sync_copy(stage, ox0.at[rows])
        pltpu.sync_copy(fz_sh.at[rows], stage)
        pltpu.sync_copy(stage, oz0.at[rows])
        pltpu.sync_copy(ft_sh.at[rows], stage)
        pltpu.sync_copy(stage, ot0.at[rows])

    @pl.when(cid == i32(1))
    def _():
        pltpu.sync_copy(fx_sh.at[rows], stage)
        pltpu.sync_copy(stage, ox1.at[rows])
        pltpu.sync_copy(fz_sh.at[rows], stage)
        pltpu.sync_copy(stage, oz1.at[rows])
        pltpu.sync_copy(ft_sh.at[rows], stage)
        pltpu.sync_copy(stage, ot1.at[rows])


def kernel(pred_raw, J_scale, connectivity, elem_lengths, prop_E, prop_A,
           prop_I22, elem_directions, F_ext, bc_disp, bc_rot):
    f32 = jnp.float32
    u_phys = pred_raw * J_scale

    conn = lax.bitcast_convert_type(connectivity, jnp.int32)[:, :, 0]
    e_pad = E_PAD - N_ELEM
    nA = jnp.concatenate([conn[:, 0], jnp.zeros((e_pad,), jnp.int32)])
    nB = jnp.concatenate([conn[:, 1], jnp.zeros((e_pad,), jnp.int32)])
    nA2 = nA.reshape(E_PAD // CHUNK, CHUNK)
    nB2 = nB.reshape(E_PAD // CHUNK, CHUNK)
    zf = jnp.zeros((e_pad,), f32)
    l_p = jnp.concatenate([elem_lengths, jnp.ones((e_pad,), f32)])
    e_p = jnp.concatenate([prop_E, zf])
    a_p = jnp.concatenate([prop_A, zf])
    i_p = jnp.concatenate([prop_I22, zf])
    c_p = jnp.concatenate([elem_directions[:, 0], zf])
    s_p = jnp.concatenate([elem_directions[:, 2], zf])

    z1 = jnp.zeros((N_PAD,), f32)
    ux = z1.at[:N_NODES].set(u_phys[:, 0])
    uz = z1.at[:N_NODES].set(u_phys[:, 1])
    th = z1.at[:N_NODES].set(u_phys[:, 2])

    mesh = plsc.VectorSubcoreMesh(core_axis_name="c", subcore_axis_name="s",
                                  num_cores=NUM_CORES,
                                  num_subcores=NUM_SUBCORES)
    sc_call = pl.kernel(
        _sc_body,
        out_type=[jax.ShapeDtypeStruct((N_PAD,), f32)] * 6,
        mesh=mesh,
        scratch_types=[
            pltpu.VMEM_SHARED((N_PAD,), f32),
            pltpu.VMEM_SHARED((N_PAD,), f32),
            pltpu.VMEM_SHARED((N_PAD,), f32),
            pltpu.VMEM_SHARED((N_PAD,), f32),
            pltpu.VMEM_SHARED((N_PAD,), f32),
            pltpu.VMEM_SHARED((N_PAD,), f32),
            pltpu.VMEM((ROWS_PER_TILE,), f32),
            pltpu.VMEM((K_PER_BATCH, CHUNK), jnp.int32),
            pltpu.VMEM((K_PER_BATCH, CHUNK), jnp.int32),
            pltpu.VMEM((BATCH,), f32),
            pltpu.VMEM((BATCH,), f32),
            pltpu.VMEM((BATCH,), f32),
            pltpu.VMEM((BATCH,), f32),
            pltpu.VMEM((BATCH,), f32),
            pltpu.VMEM((BATCH,), f32),
        ] + [pltpu.VMEM((CHUNK,), f32)] * 24 + [pltpu.SemaphoreType.DMA] * 5,
    )
    ox0, oz0, ot0, ox1, oz1, ot1 = sc_call(
        nA2, nB2, l_p, e_p, a_p, i_p, c_p, s_p, ux, uz, th, z1)

    F_internal = jnp.stack(
        [(ox0 + ox1)[:N_NODES], (oz0 + oz1)[:N_NODES],
         (ot0 + ot1)[:N_NODES]], axis=1)
    R = F_internal - F_ext
    free_disp = 1.0 - bc_disp
    free_rot = 1.0 - bc_rot
    free_mask = jnp.concatenate([free_disp, free_disp, free_rot], axis=1)
    R_normalized = R * free_mask * (J_scale * J_scale)
    n_free = jnp.clip(jnp.sum(free_mask), 1.0, None)
    m = jnp.max(jnp.abs(R_normalized))
    s = 1.0 / jnp.maximum(m, jnp.float32(1e-30))
    q = jnp.sum(jnp.square(R_normalized * s))
    loss = (q.astype(jnp.float64) * m.astype(jnp.float64) ** 2
            / n_free.astype(jnp.float64))
    return loss.astype(f32), pred_raw, u_phys


# speedup vs baseline: 1.1257x; 1.1257x over previous
import jax
import jax.numpy as jnp
from jax import lax
from jax.experimental import pallas as pl
from jax.experimental.pallas import tpu as pltpu
from jax.experimental.pallas import tpu_sc as plsc

jax.config.update("jax_enable_x64", True)

NUM_CORES = 2
NUM_SUBCORES = 16
LANES = 16
NW = NUM_CORES * NUM_SUBCORES

N_NODES = 50000
N_ELEM = 800000

ROWS_PER_TILE = 3128
N_PAD = NUM_SUBCORES * ROWS_PER_TILE

CHUNK = 128
K_PER_BATCH = 8
BATCH = K_PER_BATCH * CHUNK
N_BATCH = 25
EPW = BATCH * N_BATCH
E_PAD = EPW * NW


def _sc_body(nA_hbm, nB_hbm, l_hbm, e_hbm, a_hbm, i_hbm, c_hbm, s_hbm,
             ux_hbm, uz_hbm, th_hbm, z_hbm,
             ox0, oz0, ot0, ox1, oz1, ot1,
             ux_sh, uz_sh, th_sh, fx_sh, fz_sh, ft_sh, stage,
             nA_v, nB_v, l_v, e_v, a_v, i_v, c_v, s_v,
             uxA0, uzA0, thA0, uxB0, uzB0, thB0,
             gxA0, gzA0, gtA0, gxB0, gzB0, gtB0,
             uxA1, uzA1, thA1, uxB1, uzB1, thB1,
             gxA1, gzA1, gtA1, gxB1, gzB1, gtB1,
             semL, semG0, semG1, semS0, semS1):
    i32 = jnp.int32
    cid = lax.axis_index("c")
    sid = lax.axis_index("s")
    wid = cid * i32(NUM_SUBCORES) + sid

    row0 = pl.multiple_of(sid * i32(ROWS_PER_TILE), 8)
    rows = pl.ds(row0, ROWS_PER_TILE)
    pltpu.sync_copy(ux_hbm.at[rows], stage)
    pltpu.sync_copy(stage, ux_sh.at[rows])
    pltpu.sync_copy(uz_hbm.at[rows], stage)
    pltpu.sync_copy(stage, uz_sh.at[rows])
    pltpu.sync_copy(th_hbm.at[rows], stage)
    pltpu.sync_copy(stage, th_sh.at[rows])
    pltpu.sync_copy(z_hbm.at[rows], stage)
    pltpu.sync_copy(stage, fx_sh.at[rows])
    pltpu.sync_copy(stage, fz_sh.at[rows])
    pltpu.sync_copy(stage, ft_sh.at[rows])
    plsc.subcore_barrier()

    ebase = wid * i32(EPW)
    rbase = wid * i32(EPW // CHUNK)

    def compute_chunk(j, uxA_v, uzA_v, thA_v, uxB_v, uzB_v, thB_v,
                      gxA_v, gzA_v, gtA_v, gxB_v, gzB_v, gtB_v):
        def step(i, carry3):
            sb = pl.ds(j * i32(CHUNK) + i * i32(LANES), LANES)
            sc = pl.ds(i * i32(LANES), LANES)
            uxA = uxA_v[sc]
            uzA = uzA_v[sc]
            thA = thA_v[sc]
            uxB = uxB_v[sc]
            uzB = uzB_v[sc]
            thB = thB_v[sc]
            el = l_v[sb]
            ee = e_v[sb]
            aa = a_v[sb]
            ii = i_v[sb]
            cc = c_v[sb]
            ss = s_v[sb]

            inv_l = 1.0 / el
            ea_l = ee * aa * inv_l
            ei_l = ee * ii * inv_l
            ei_l2 = ei_l * inv_l
            ei_l3 = ei_l2 * inv_l

            u_loc_d = cc * (uxA - uxB) + ss * (uzA - uzB)
            wA = cc * uzA - ss * uxA
            wB = cc * uzB - ss * uxB
            dw = wA - wB
            thAl = -thA
            thBl = -thB
            sth = thAl + thBl

            f0 = ea_l * u_loc_d
            f1 = 12.0 * ei_l3 * dw + 6.0 * ei_l2 * sth
            b_dw = 6.0 * ei_l2 * dw
            f2 = b_dw + 4.0 * ei_l * thAl + 2.0 * ei_l * thBl
            f5 = b_dw + 2.0 * ei_l * thAl + 4.0 * ei_l * thBl

            gAx = cc * f0 - ss * f1
            gAz = ss * f0 + cc * f1
            gxA_v[sc] = gAx
            gzA_v[sc] = gAz
            gtA_v[sc] = -f2
            gxB_v[sc] = -gAx
            gzB_v[sc] = -gAz
            gtB_v[sc] = -f5
            return carry3

        lax.fori_loop(i32(0), i32(CHUNK // LANES), step, i32(0),
                      unroll=False)

    def batch_body(bi, carry):
        eb = pl.multiple_of(ebase + bi * i32(BATCH), 8)
        rb = pl.multiple_of(rbase + bi * i32(K_PER_BATCH), 8)
        lds = [
            pltpu.async_copy(nA_hbm.at[pl.ds(rb, K_PER_BATCH)], nA_v, semL),
            pltpu.async_copy(nB_hbm.at[pl.ds(rb, K_PER_BATCH)], nB_v, semL),
            pltpu.async_copy(l_hbm.at[pl.ds(eb, BATCH)], l_v, semL),
            pltpu.async_copy(e_hbm.at[pl.ds(eb, BATCH)], e_v, semL),
            pltpu.async_copy(a_hbm.at[pl.ds(eb, BATCH)], a_v, semL),
            pltpu.async_copy(i_hbm.at[pl.ds(eb, BATCH)], i_v, semL),
            pltpu.async_copy(c_hbm.at[pl.ds(eb, BATCH)], c_v, semL),
            pltpu.async_copy(s_hbm.at[pl.ds(eb, BATCH)], s_v, semL),
        ]
        for cp in lds:
            cp.wait()

        def pair_body(p, carry2):
            j0 = p * i32(2)
            j1 = j0 + i32(1)
            idxA0 = nA_v.at[j0]
            idxB0 = nB_v.at[j0]
            idxA1 = nA_v.at[j1]
            idxB1 = nB_v.at[j1]
            ga = [
                pltpu.async_copy(ux_sh.at[idxA0], uxA0, semG0),
                pltpu.async_copy(uz_sh.at[idxA0], uzA0, semG0),
                pltpu.async_copy(th_sh.at[idxA0], thA0, semG0),
                pltpu.async_copy(ux_sh.at[idxB0], uxB0, semG0),
                pltpu.async_copy(uz_sh.at[idxB0], uzB0, semG0),
                pltpu.async_copy(th_sh.at[idxB0], thB0, semG0),
            ]
            gb = [
                pltpu.async_copy(ux_sh.at[idxA1], uxA1, semG1),
                pltpu.async_copy(uz_sh.at[idxA1], uzA1, semG1),
                pltpu.async_copy(th_sh.at[idxA1], thA1, semG1),
                pltpu.async_copy(ux_sh.at[idxB1], uxB1, semG1),
                pltpu.async_copy(uz_sh.at[idxB1], uzB1, semG1),
                pltpu.async_copy(th_sh.at[idxB1], thB1, semG1),
            ]
            for cp in ga:
                cp.wait()
            compute_chunk(j0, uxA0, uzA0, thA0, uxB0, uzB0, thB0,
                          gxA0, gzA0, gtA0, gxB0, gzB0, gtB0)
            sa = [
                pltpu.async_copy(gxA0, fx_sh.at[idxA0], semS0, add=True),
                pltpu.async_copy(gzA0, fz_sh.at[idxA0], semS0, add=True),
                pltpu.async_copy(gtA0, ft_sh.at[idxA0], semS0, add=True),
                pltpu.async_copy(gxB0, fx_sh.at[idxB0], semS0, add=True),
                pltpu.async_copy(gzB0, fz_sh.at[idxB0], semS0, add=True),
                pltpu.async_copy(gtB0, ft_sh.at[idxB0], semS0, add=True),
            ]
            for cp in gb:
                cp.wait()
            compute_chunk(j1, uxA1, uzA1, thA1, uxB1, uzB1, thB1,
                          gxA1, gzA1, gtA1, gxB1, gzB1, gtB1)
            sb_ = [
                pltpu.async_copy(gxA1, fx_sh.at[idxA1], semS1, add=True),
                pltpu.async_copy(gzA1, fz_sh.at[idxA1], semS1, add=True),
                pltpu.async_copy(gtA1, ft_sh.at[idxA1], semS1, add=True),
                pltpu.async_copy(gxB1, fx_sh.at[idxB1], semS1, add=True),
                pltpu.async_copy(gzB1, fz_sh.at[idxB1], semS1, add=True),
                pltpu.async_copy(gtB1, ft_sh.at[idxB1], semS1, add=True),
            ]
            for cp in sa:
                cp.wait()
            for cp in sb_:
                cp.wait()
            return carry2

        lax.fori_loop(i32(0), i32(K_PER_BATCH // 2), pair_body, i32(0),
                      unroll=False)
        return carry

    lax.fori_loop(i32(0), i32(N_BATCH), batch_body, i32(0), unroll=False)

    plsc.subcore_barrier()

    @pl.when(cid == i32(0))
    def _():
        pltpu.sync_copy(fx_sh.at[rows], stage)
        pltpu.sync_copy(stage, ox0.at[rows])
        pltpu.sync_copy(fz_sh.at[rows], stage)
        pltpu.sync_copy(stage, oz0.at[rows])
        pltpu.sync_copy(ft_sh.at[rows], stage)
        pltpu.sync_copy(stage, ot0.at[rows])

    @pl.when(cid == i32(1))
    def _():
        pltpu.sync_copy(fx_sh.at[rows], stage)
        pltpu.sync_copy(stage, ox1.at[rows])
        pltpu.sync_copy(fz_sh.at[rows], stage)
        pltpu.sync_copy(stage, oz1.at[rows])
        pltpu.sync_copy(ft_sh.at[rows], stage)
        pltpu.sync_copy(stage, ot1.at[rows])


def kernel(pred_raw, J_scale, connectivity, elem_lengths, prop_E, prop_A,
           prop_I22, elem_directions, F_ext, bc_disp, bc_rot):
    f32 = jnp.float32
    u_phys = pred_raw * J_scale

    conn = connectivity.astype(jnp.int32)
    e_pad = E_PAD - N_ELEM
    nA = jnp.concatenate([conn[:, 0], jnp.zeros((e_pad,), jnp.int32)])
    nB = jnp.concatenate([conn[:, 1], jnp.zeros((e_pad,), jnp.int32)])
    nA2 = nA.reshape(E_PAD // CHUNK, CHUNK)
    nB2 = nB.reshape(E_PAD // CHUNK, CHUNK)
    zf = jnp.zeros((e_pad,), f32)
    l_p = jnp.concatenate([elem_lengths, jnp.ones((e_pad,), f32)])
    e_p = jnp.concatenate([prop_E, zf])
    a_p = jnp.concatenate([prop_A, zf])
    i_p = jnp.concatenate([prop_I22, zf])
    c_p = jnp.concatenate([elem_directions[:, 0], zf])
    s_p = jnp.concatenate([elem_directions[:, 2], zf])

    z1 = jnp.zeros((N_PAD,), f32)
    ux = z1.at[:N_NODES].set(u_phys[:, 0])
    uz = z1.at[:N_NODES].set(u_phys[:, 1])
    th = z1.at[:N_NODES].set(u_phys[:, 2])

    mesh = plsc.VectorSubcoreMesh(core_axis_name="c", subcore_axis_name="s",
                                  num_cores=NUM_CORES,
                                  num_subcores=NUM_SUBCORES)
    sc_call = pl.kernel(
        _sc_body,
        out_type=[jax.ShapeDtypeStruct((N_PAD,), f32)] * 6,
        mesh=mesh,
        scratch_types=[
            pltpu.VMEM_SHARED((N_PAD,), f32),
            pltpu.VMEM_SHARED((N_PAD,), f32),
            pltpu.VMEM_SHARED((N_PAD,), f32),
            pltpu.VMEM_SHARED((N_PAD,), f32),
            pltpu.VMEM_SHARED((N_PAD,), f32),
            pltpu.VMEM_SHARED((N_PAD,), f32),
            pltpu.VMEM((ROWS_PER_TILE,), f32),
            pltpu.VMEM((K_PER_BATCH, CHUNK), jnp.int32),
            pltpu.VMEM((K_PER_BATCH, CHUNK), jnp.int32),
            pltpu.VMEM((BATCH,), f32),
            pltpu.VMEM((BATCH,), f32),
            pltpu.VMEM((BATCH,), f32),
            pltpu.VMEM((BATCH,), f32),
            pltpu.VMEM((BATCH,), f32),
            pltpu.VMEM((BATCH,), f32),
        ] + [pltpu.VMEM((CHUNK,), f32)] * 24 + [pltpu.SemaphoreType.DMA] * 5,
    )
    ox0, oz0, ot0, ox1, oz1, ot1 = sc_call(
        nA2, nB2, l_p, e_p, a_p, i_p, c_p, s_p, ux, uz, th, z1)

    F_internal = jnp.stack(
        [(ox0 + ox1)[:N_NODES], (oz0 + oz1)[:N_NODES],
         (ot0 + ot1)[:N_NODES]], axis=1)
    R = F_internal - F_ext
    free_disp = 1.0 - bc_disp
    free_rot = 1.0 - bc_rot
    free_mask = jnp.concatenate([free_disp, free_disp, free_rot], axis=1)
    R_normalized = R * free_mask * (J_scale * J_scale)
    n_free = jnp.clip(jnp.sum(free_mask), 1.0, None)
    m = jnp.max(jnp.abs(R_normalized))
    s = 1.0 / jnp.maximum(m, jnp.float32(1e-30))
    q = jnp.sum(jnp.square(R_normalized * s))
    loss = (q.astype(jnp.float64) * m.astype(jnp.float64) ** 2
            / n_free.astype(jnp.float64))
    return loss.astype(f32), pred_raw, u_phys


# speedup vs baseline: 1.2719x; 1.1298x over previous
import jax
import jax.numpy as jnp
from jax import lax
from jax.experimental import pallas as pl
from jax.experimental.pallas import tpu as pltpu
from jax.experimental.pallas import tpu_sc as plsc

jax.config.update("jax_enable_x64", True)

NUM_CORES = 2
NUM_SUBCORES = 16
LANES = 16
NW = NUM_CORES * NUM_SUBCORES

N_NODES = 50000
N_ELEM = 800000

ROWS_PER_TILE = 3128
N_PAD = NUM_SUBCORES * ROWS_PER_TILE

CHUNK = 128
K_PER_BATCH = 8
BATCH = K_PER_BATCH * CHUNK
N_BATCH = 25
EPW = BATCH * N_BATCH
E_PAD = EPW * NW

NSET = 4


def _sc_body(nA_hbm, nB_hbm, l_hbm, e_hbm, a_hbm, i_hbm, c_hbm, s_hbm,
             ux_hbm, uz_hbm, th_hbm, z_hbm,
             ox0, oz0, ot0, ox1, oz1, ot1, *scr):
    ux_sh, uz_sh, th_sh, fx_sh, fz_sh, ft_sh, stage = scr[:7]
    nA_v, nB_v, l_v, e_v, a_v, i_v, c_v, s_v = scr[7:15]
    bufs = [scr[15 + 12 * k:15 + 12 * (k + 1)] for k in range(NSET)]
    semL = scr[15 + 12 * NSET]
    semG = scr[16 + 12 * NSET:16 + 13 * NSET]
    semS = scr[16 + 13 * NSET:16 + 14 * NSET]

    i32 = jnp.int32
    cid = lax.axis_index("c")
    sid = lax.axis_index("s")
    wid = cid * i32(NUM_SUBCORES) + sid

    row0 = pl.multiple_of(sid * i32(ROWS_PER_TILE), 8)
    rows = pl.ds(row0, ROWS_PER_TILE)
    pltpu.sync_copy(ux_hbm.at[rows], stage)
    pltpu.sync_copy(stage, ux_sh.at[rows])
    pltpu.sync_copy(uz_hbm.at[rows], stage)
    pltpu.sync_copy(stage, uz_sh.at[rows])
    pltpu.sync_copy(th_hbm.at[rows], stage)
    pltpu.sync_copy(stage, th_sh.at[rows])
    pltpu.sync_copy(z_hbm.at[rows], stage)
    pltpu.sync_copy(stage, fx_sh.at[rows])
    pltpu.sync_copy(stage, fz_sh.at[rows])
    pltpu.sync_copy(stage, ft_sh.at[rows])
    plsc.subcore_barrier()

    ebase = wid * i32(EPW)
    rbase = wid * i32(EPW // CHUNK)

    def compute_chunk(j, uxA_v, uzA_v, thA_v, uxB_v, uzB_v, thB_v,
                      gxA_v, gzA_v, gtA_v, gxB_v, gzB_v, gtB_v):
        def step(i, carry3):
            sb = pl.ds(j * i32(CHUNK) + i * i32(LANES), LANES)
            sc = pl.ds(i * i32(LANES), LANES)
            uxA = uxA_v[sc]
            uzA = uzA_v[sc]
            thA = thA_v[sc]
            uxB = uxB_v[sc]
            uzB = uzB_v[sc]
            thB = thB_v[sc]
            el = l_v[sb]
            ee = e_v[sb]
            aa = a_v[sb]
            ii = i_v[sb]
            cc = c_v[sb]
            ss = s_v[sb]

            inv_l = 1.0 / el
            ea_l = ee * aa * inv_l
            ei_l = ee * ii * inv_l
            ei_l2 = ei_l * inv_l
            ei_l3 = ei_l2 * inv_l

            u_loc_d = cc * (uxA - uxB) + ss * (uzA - uzB)
            wA = cc * uzA - ss * uxA
            wB = cc * uzB - ss * uxB
            dw = wA - wB
            thAl = -thA
            thBl = -thB
            sth = thAl + thBl

            f0 = ea_l * u_loc_d
            f1 = 12.0 * ei_l3 * dw + 6.0 * ei_l2 * sth
            b_dw = 6.0 * ei_l2 * dw
            f2 = b_dw + 4.0 * ei_l * thAl + 2.0 * ei_l * thBl
            f5 = b_dw + 2.0 * ei_l * thAl + 4.0 * ei_l * thBl

            gAx = cc * f0 - ss * f1
            gAz = ss * f0 + cc * f1
            gxA_v[sc] = gAx
            gzA_v[sc] = gAz
            gtA_v[sc] = -f2
            gxB_v[sc] = -gAx
            gzB_v[sc] = -gAz
            gtB_v[sc] = -f5
            return carry3

        lax.fori_loop(i32(0), i32(CHUNK // LANES), step, i32(0),
                      unroll=False)

    def batch_body(bi, carry):
        eb = pl.multiple_of(ebase + bi * i32(BATCH), 8)
        rb = pl.multiple_of(rbase + bi * i32(K_PER_BATCH), 8)
        lds = [
            pltpu.async_copy(nA_hbm.at[pl.ds(rb, K_PER_BATCH)], nA_v, semL),
            pltpu.async_copy(nB_hbm.at[pl.ds(rb, K_PER_BATCH)], nB_v, semL),
            pltpu.async_copy(l_hbm.at[pl.ds(eb, BATCH)], l_v, semL),
            pltpu.async_copy(e_hbm.at[pl.ds(eb, BATCH)], e_v, semL),
            pltpu.async_copy(a_hbm.at[pl.ds(eb, BATCH)], a_v, semL),
            pltpu.async_copy(i_hbm.at[pl.ds(eb, BATCH)], i_v, semL),
            pltpu.async_copy(c_hbm.at[pl.ds(eb, BATCH)], c_v, semL),
            pltpu.async_copy(s_hbm.at[pl.ds(eb, BATCH)], s_v, semL),
        ]
        for cp in lds:
            cp.wait()

        def issue_gather(k):
            b = bufs[k % NSET]
            idxA = nA_v.at[jnp.int32(k)]
            idxB = nB_v.at[jnp.int32(k)]
            sem = semG[k % NSET]
            return (idxA, idxB, [
                pltpu.async_copy(ux_sh.at[idxA], b[0], sem),
                pltpu.async_copy(uz_sh.at[idxA], b[1], sem),
                pltpu.async_copy(th_sh.at[idxA], b[2], sem),
                pltpu.async_copy(ux_sh.at[idxB], b[3], sem),
                pltpu.async_copy(uz_sh.at[idxB], b[4], sem),
                pltpu.async_copy(th_sh.at[idxB], b[5], sem),
            ])

        gops = [None] * K_PER_BATCH
        sops = [None] * K_PER_BATCH
        for k in range(NSET):
            gops[k] = issue_gather(k)
        for k in range(K_PER_BATCH):
            idxA, idxB, ga = gops[k]
            for cp in ga:
                cp.wait()
            if k >= NSET:
                for cp in sops[k - NSET]:
                    cp.wait()
            b = bufs[k % NSET]
            compute_chunk(k, *b)
            sem = semS[k % NSET]
            sops[k] = [
                pltpu.async_copy(b[6], fx_sh.at[idxA], sem, add=True),
                pltpu.async_copy(b[7], fz_sh.at[idxA], sem, add=True),
                pltpu.async_copy(b[8], ft_sh.at[idxA], sem, add=True),
                pltpu.async_copy(b[9], fx_sh.at[idxB], sem, add=True),
                pltpu.async_copy(b[10], fz_sh.at[idxB], sem, add=True),
                pltpu.async_copy(b[11], ft_sh.at[idxB], sem, add=True),
            ]
            if k + NSET < K_PER_BATCH:
                gops[k + NSET] = issue_gather(k + NSET)
        for k in range(K_PER_BATCH - NSET, K_PER_BATCH):
            for cp in sops[k]:
                cp.wait()
        return carry

    lax.fori_loop(i32(0), i32(N_BATCH), batch_body, i32(0), unroll=False)

    plsc.subcore_barrier()

    @pl.when(cid == i32(0))
    def _():
        pltpu.sync_copy(fx_sh.at[rows], stage)
        pltpu.sync_copy(stage, ox0.at[rows])
        pltpu.sync_copy(fz_sh.at[rows], stage)
        pltpu.sync_copy(stage, oz0.at[rows])
        pltpu.sync_copy(ft_sh.at[rows], stage)
        pltpu.sync_copy(stage, ot0.at[rows])

    @pl.when(cid == i32(1))
    def _():
        pltpu.sync_copy(fx_sh.at[rows], stage)
        pltpu.sync_copy(stage, ox1.at[rows])
        pltpu.sync_copy(fz_sh.at[rows], stage)
        pltpu.sync_copy(stage, oz1.at[rows])
        pltpu.sync_copy(ft_sh.at[rows], stage)
        pltpu.sync_copy(stage, ot1.at[rows])


def kernel(pred_raw, J_scale, connectivity, elem_lengths, prop_E, prop_A,
           prop_I22, elem_directions, F_ext, bc_disp, bc_rot):
    f32 = jnp.float32
    u_phys = pred_raw * J_scale

    conn = connectivity.astype(jnp.int32)
    e_pad = E_PAD - N_ELEM
    nA = jnp.concatenate([conn[:, 0], jnp.zeros((e_pad,), jnp.int32)])
    nB = jnp.concatenate([conn[:, 1], jnp.zeros((e_pad,), jnp.int32)])
    nA2 = nA.reshape(E_PAD // CHUNK, CHUNK)
    nB2 = nB.reshape(E_PAD // CHUNK, CHUNK)
    zf = jnp.zeros((e_pad,), f32)
    l_p = jnp.concatenate([elem_lengths, jnp.ones((e_pad,), f32)])
    e_p = jnp.concatenate([prop_E, zf])
    a_p = jnp.concatenate([prop_A, zf])
    i_p = jnp.concatenate([prop_I22, zf])
    c_p = jnp.concatenate([elem_directions[:, 0], zf])
    s_p = jnp.concatenate([elem_directions[:, 2], zf])

    z1 = jnp.zeros((N_PAD,), f32)
    ux = z1.at[:N_NODES].set(u_phys[:, 0])
    uz = z1.at[:N_NODES].set(u_phys[:, 1])
    th = z1.at[:N_NODES].set(u_phys[:, 2])

    mesh = plsc.VectorSubcoreMesh(core_axis_name="c", subcore_axis_name="s",
                                  num_cores=NUM_CORES,
                                  num_subcores=NUM_SUBCORES)
    sc_call = pl.kernel(
        _sc_body,
        out_type=[jax.ShapeDtypeStruct((N_PAD,), f32)] * 6,
        mesh=mesh,
        scratch_types=[
            pltpu.VMEM_SHARED((N_PAD,), f32),
            pltpu.VMEM_SHARED((N_PAD,), f32),
            pltpu.VMEM_SHARED((N_PAD,), f32),
            pltpu.VMEM_SHARED((N_PAD,), f32),
            pltpu.VMEM_SHARED((N_PAD,), f32),
            pltpu.VMEM_SHARED((N_PAD,), f32),
            pltpu.VMEM((ROWS_PER_TILE,), f32),
            pltpu.VMEM((K_PER_BATCH, CHUNK), jnp.int32),
            pltpu.VMEM((K_PER_BATCH, CHUNK), jnp.int32),
            pltpu.VMEM((BATCH,), f32),
            pltpu.VMEM((BATCH,), f32),
            pltpu.VMEM((BATCH,), f32),
            pltpu.VMEM((BATCH,), f32),
            pltpu.VMEM((BATCH,), f32),
            pltpu.VMEM((BATCH,), f32),
        ] + [pltpu.VMEM((CHUNK,), f32)] * (12 * NSET)
          + [pltpu.SemaphoreType.DMA] * (1 + 2 * NSET),
    )
    ox0, oz0, ot0, ox1, oz1, ot1 = sc_call(
        nA2, nB2, l_p, e_p, a_p, i_p, c_p, s_p, ux, uz, th, z1)

    F_internal = jnp.stack(
        [(ox0 + ox1)[:N_NODES], (oz0 + oz1)[:N_NODES],
         (ot0 + ot1)[:N_NODES]], axis=1)
    R = F_internal - F_ext
    free_disp = 1.0 - bc_disp
    free_rot = 1.0 - bc_rot
    free_mask = jnp.concatenate([free_disp, free_disp, free_rot], axis=1)
    R_normalized = R * free_mask * (J_scale * J_scale)
    n_free = jnp.clip(jnp.sum(free_mask), 1.0, None)
    m = jnp.max(jnp.abs(R_normalized))
    s = 1.0 / jnp.maximum(m, jnp.float32(1e-30))
    q = jnp.sum(jnp.square(R_normalized * s))
    loss = (q.astype(jnp.float64) * m.astype(jnp.float64) ** 2
            / n_free.astype(jnp.float64))
    return loss.astype(f32), pred_raw, u_phys
